# baseline (speedup 1.0000x reference)
import functools

import jax
import jax.numpy as jnp
from jax import lax
from jax.experimental import pallas as pl
from jax.experimental.pallas import tpu as pltpu
from jax.experimental.pallas import tpu_sc as plsc

T, D, E, K, H = 2048, 1024, 8, 2, 4096
BM = 256
HTILE = 512
HT = H // HTILE
P = K * T + E * BM
NB = P // BM
NC, NS = 2, 16
NW = NC * NS
ROWS_G = P // NW
CH = 64
TW = T // NW
CT = 32


def _router(xf, Wg):
    gate_logits = xf @ Wg.T
    gate_probs = jax.nn.softmax(gate_logits, axis=-1)
    rw, ei = lax.top_k(gate_probs, K)
    rw = rw / rw.sum(axis=-1, keepdims=True)
    return gate_probs, rw, ei


def _dispatch(rw, ei):
    ee = ei.reshape(-1)
    onehot = (ee[:, None] == jnp.arange(E, dtype=ee.dtype)[None, :])
    incl = jnp.cumsum(onehot.astype(jnp.int32), axis=0)
    rank = jnp.take_along_axis(incl, ee[:, None].astype(jnp.int32), axis=1)[:, 0] - 1
    counts = incl[-1]
    padded = ((counts + BM - 1) // BM) * BM
    ends = jnp.cumsum(padded)
    starts = ends - padded
    posf = (starts[ee] + rank).astype(jnp.int32)
    total = ends[-1]
    pair_tok = (jnp.arange(K * T, dtype=jnp.int32) // K)
    src = jnp.zeros((P,), jnp.int32).at[posf].set(pair_tok)
    wslot = jnp.zeros((P,), jnp.float32).at[posf].set(rw.reshape(-1))
    bstart = jnp.arange(NB, dtype=jnp.int32) * BM
    act = (bstart < total).astype(jnp.int32)
    ebf = jnp.sum(bstart[:, None] >= ends[None, :], axis=1).astype(jnp.int32)
    n_act = total // BM
    eb_last = ebf[jnp.maximum(n_act - 1, 0)]
    eb = jnp.where(act == 1, ebf, eb_last).astype(jnp.int32)
    pos0 = posf[0::2]
    pos1 = posf[1::2]
    return src, wslot, eb, act, pos0, pos1, counts


def _mlp_body(eb_ref, act_ref, x_ref, wfc_ref, wpr_ref, w_ref, o_ref):
    b = pl.program_id(0)
    ht = pl.program_id(1)

    @pl.when(act_ref[b] == 1)
    def _():
        h = lax.dot_general(x_ref[...], wfc_ref[0],
                            (((1,), (1,)), ((), ())),
                            preferred_element_type=jnp.float32)
        h = jnp.square(jnp.maximum(h, 0.0))
        y = lax.dot_general(h, wpr_ref[0],
                            (((1,), (1,)), ((), ())),
                            preferred_element_type=jnp.float32)

        @pl.when(ht == 0)
        def _():
            o_ref[...] = y

        @pl.when(ht > 0)
        def _():
            o_ref[...] += y

        @pl.when(ht == HT - 1)
        def _():
            o_ref[...] = o_ref[...] * w_ref[0]


def _mlp_call(eb, act, xs, Wfc, Wproj, w3, interpret=False):
    return pl.pallas_call(
        _mlp_body,
        grid_spec=pltpu.PrefetchScalarGridSpec(
            num_scalar_prefetch=2,
            grid=(NB, HT),
            in_specs=[
                pl.BlockSpec((BM, D), lambda b, ht, eb_r, act_r: (b, 0)),
                pl.BlockSpec((1, HTILE, D),
                             lambda b, ht, eb_r, act_r: (eb_r[b], ht, 0)),
                pl.BlockSpec((1, D, HTILE),
                             lambda b, ht, eb_r, act_r: (eb_r[b], 0, ht)),
                pl.BlockSpec((1, BM, 1), lambda b, ht, eb_r, act_r: (b, 0, 0)),
            ],
            out_specs=pl.BlockSpec((BM, D), lambda b, ht, eb_r, act_r: (b, 0)),
        ),
        out_shape=jax.ShapeDtypeStruct((P, D), jnp.float32),
        interpret=interpret,
    )(eb, act, xs, Wfc, Wproj, w3)


def _sc_gather_call(src, xf):
    mesh = plsc.VectorSubcoreMesh(core_axis_name="c", subcore_axis_name="s")

    @functools.partial(
        pl.kernel,
        mesh=mesh,
        out_type=jax.ShapeDtypeStruct((P, D), jnp.float32),
        scratch_types=[
            pltpu.VMEM((CH,), jnp.int32),
            pltpu.VMEM((CH, D), jnp.float32),
            pltpu.SemaphoreType.DMA,
        ],
    )
    def k(src_hbm, x_hbm, out_hbm, idx_v, rows_v, sem):
        wid = lax.axis_index("s") * NC + lax.axis_index("c")
        base = wid * ROWS_G
        for c in range(ROWS_G // CH):
            off = base + c * CH
            pltpu.sync_copy(src_hbm.at[pl.ds(off, CH)], idx_v)
            pltpu.async_copy(x_hbm.at[idx_v], rows_v, sem).wait()
            pltpu.sync_copy(rows_v, out_hbm.at[pl.ds(off, CH)])

    return k(src, xf)


def _sc_combine_call(pos0, pos1, ys):
    mesh = plsc.VectorSubcoreMesh(core_axis_name="c", subcore_axis_name="s")

    @functools.partial(
        pl.kernel,
        mesh=mesh,
        out_type=jax.ShapeDtypeStruct((T, D), jnp.float32),
        scratch_types=[
            pltpu.VMEM((CT,), jnp.int32),
            pltpu.VMEM((CT,), jnp.int32),
            pltpu.VMEM((CT, D), jnp.float32),
            pltpu.VMEM((CT, D), jnp.float32),
            pltpu.SemaphoreType.DMA,
            pltpu.SemaphoreType.DMA,
        ],
    )
    def k(p0_hbm, p1_hbm, ys_hbm, out_hbm, p0_v, p1_v, a_v, b_v, s0, s1):
        wid = lax.axis_index("s") * NC + lax.axis_index("c")
        base = wid * TW
        for c in range(TW // CT):
            tb = base + c * CT
            pltpu.sync_copy(p0_hbm.at[pl.ds(tb, CT)], p0_v)
            pltpu.sync_copy(p1_hbm.at[pl.ds(tb, CT)], p1_v)
            cp0 = pltpu.async_copy(ys_hbm.at[p0_v], a_v, s0)
            cp1 = pltpu.async_copy(ys_hbm.at[p1_v], b_v, s1)
            cp0.wait()
            cp1.wait()

            def row(r, carry):
                def col(j, carry2):
                    sl = pl.ds(j * 16, 16)
                    a_v[r, sl] = a_v[r, sl] + b_v[r, sl]
                    return carry2
                return lax.fori_loop(0, D // 16, col, carry)

            lax.fori_loop(0, CT, row, 0)
            pltpu.sync_copy(a_v, out_hbm.at[pl.ds(tb, CT)])

    return k(pos0, pos1, ys)


def kernel(x, Wg, Wfc, Wproj):
    Bv, Tv, Dv = x.shape
    xf = x.reshape(Tv * Bv, Dv)
    gate_probs, rw, ei = _router(xf, Wg)
    src, wslot, eb, act, pos0, pos1, counts = _dispatch(rw, ei)
    xs = _sc_gather_call(src, xf)
    ys = _mlp_call(eb, act, xs, Wfc, Wproj, wslot.reshape(NB, BM, 1))
    out = _sc_combine_call(pos0, pos1, ys)
    freq = counts.astype(jnp.float32) / T
    balance_loss = (gate_probs.mean(axis=0) * freq).sum() * E
    return out.reshape(Bv, Tv, Dv), balance_loss


# speedup vs baseline: 2.1434x; 2.1434x over previous
import functools

import jax
import jax.numpy as jnp
from jax import lax
from jax.experimental import pallas as pl
from jax.experimental.pallas import tpu as pltpu
from jax.experimental.pallas import tpu_sc as plsc

T, D, E, K, H = 2048, 1024, 8, 2, 4096
BM = 256
HTILE = 512
HT = H // HTILE
P = K * T + E * BM
NB = P // BM
NC, NS = 2, 16
NW = NC * NS
ROWS_G = P // NW
CH = 64
TW = T // NW
CT = 32


def _router(xf, Wg):
    gate_logits = xf @ Wg.T
    gate_probs = jax.nn.softmax(gate_logits, axis=-1)
    rw, ei = lax.top_k(gate_probs, K)
    rw = rw / rw.sum(axis=-1, keepdims=True)
    return gate_probs, rw, ei


def _dispatch(rw, ei):
    ee = ei.reshape(-1)
    onehot = (ee[:, None] == jnp.arange(E, dtype=ee.dtype)[None, :])
    incl = jnp.cumsum(onehot.astype(jnp.int32), axis=0)
    rank = jnp.take_along_axis(incl, ee[:, None].astype(jnp.int32), axis=1)[:, 0] - 1
    counts = incl[-1]
    padded = ((counts + BM - 1) // BM) * BM
    ends = jnp.cumsum(padded)
    starts = ends - padded
    posf = (starts[ee] + rank).astype(jnp.int32)
    total = ends[-1]
    wslot = jnp.zeros((P,), jnp.float32).at[posf].set(rw.reshape(-1))
    bstart = jnp.arange(NB, dtype=jnp.int32) * BM
    act = (bstart < total).astype(jnp.int32)
    ebf = jnp.sum(bstart[:, None] >= ends[None, :], axis=1).astype(jnp.int32)
    n_act = total // BM
    eb_last = ebf[jnp.maximum(n_act - 1, 0)]
    eb = jnp.where(act == 1, ebf, eb_last).astype(jnp.int32)
    prev_eb = jnp.concatenate([eb[:1] - 1, eb[:-1]])
    chg = (act * (eb != prev_eb)).astype(jnp.int32)
    pos0 = posf[0::2]
    pos1 = posf[1::2]
    return wslot, eb, act, chg, pos0, pos1, counts


def _mlp_body(eb_ref, act_ref, chg_ref, x_ref, wfc_ref, wpr_hbm, w_ref, o_ref,
              h_ref, wp_v, sem):
    b = pl.program_id(0)

    @pl.when(chg_ref[b] == 1)
    def _():
        pltpu.make_async_copy(wpr_hbm.at[eb_ref[b]], wp_v, sem).start()

    @pl.when(act_ref[b] == 1)
    def _():
        x = x_ref[...]
        for t in range(HT):
            h = lax.dot_general(x, wfc_ref[0, pl.ds(t * HTILE, HTILE), :],
                                (((1,), (1,)), ((), ())),
                                preferred_element_type=jnp.float32)
            h_ref[:, pl.ds(t * HTILE, HTILE)] = jnp.square(jnp.maximum(h, 0.0))

        @pl.when(chg_ref[b] == 1)
        def _():
            pltpu.make_async_copy(wpr_hbm.at[eb_ref[b]], wp_v, sem).wait()

        y = lax.dot_general(h_ref[...], wp_v[...],
                            (((1,), (1,)), ((), ())),
                            preferred_element_type=jnp.float32)
        o_ref[...] = y * w_ref[0]


def _mlp_call(eb, act, chg, xs, Wfc, Wproj, w3, interpret=False):
    return pl.pallas_call(
        _mlp_body,
        grid_spec=pltpu.PrefetchScalarGridSpec(
            num_scalar_prefetch=3,
            grid=(NB,),
            in_specs=[
                pl.BlockSpec((BM, D), lambda b, eb_r, act_r, chg_r: (b, 0)),
                pl.BlockSpec((1, H, D),
                             lambda b, eb_r, act_r, chg_r: (eb_r[b], 0, 0)),
                pl.BlockSpec(memory_space=pl.ANY),
                pl.BlockSpec((1, BM, 1),
                             lambda b, eb_r, act_r, chg_r: (b, 0, 0)),
            ],
            out_specs=pl.BlockSpec((BM, D),
                                   lambda b, eb_r, act_r, chg_r: (b, 0)),
            scratch_shapes=[
                pltpu.VMEM((BM, H), jnp.float32),
                pltpu.VMEM((D, H), jnp.float32),
                pltpu.SemaphoreType.DMA,
            ],
        ),
        out_shape=jax.ShapeDtypeStruct((P, D), jnp.float32),
        compiler_params=pltpu.CompilerParams(
            vmem_limit_bytes=62 * 1024 * 1024),
        interpret=interpret,
    )(eb, act, chg, xs, Wfc, Wproj, w3)


def _sc_dispatch_call(pos0, pos1, xf):
    mesh = plsc.VectorSubcoreMesh(core_axis_name="c", subcore_axis_name="s")

    @functools.partial(
        pl.kernel,
        mesh=mesh,
        out_type=jax.ShapeDtypeStruct((P, D), jnp.float32),
        scratch_types=[
            pltpu.VMEM((TW,), jnp.int32),
            pltpu.VMEM((TW,), jnp.int32),
            pltpu.VMEM((TW, D), jnp.float32),
            pltpu.SemaphoreType.DMA,
            pltpu.SemaphoreType.DMA,
        ],
    )
    def k(p0_hbm, p1_hbm, x_hbm, out_hbm, p0_v, p1_v, rows_v, s0, s1):
        wid = lax.axis_index("s") * NC + lax.axis_index("c")
        tb = wid * TW
        pltpu.sync_copy(p0_hbm.at[pl.ds(tb, TW)], p0_v)
        pltpu.sync_copy(p1_hbm.at[pl.ds(tb, TW)], p1_v)
        pltpu.sync_copy(x_hbm.at[pl.ds(tb, TW)], rows_v)
        c0 = pltpu.async_copy(rows_v, out_hbm.at[p0_v], s0)
        c1 = pltpu.async_copy(rows_v, out_hbm.at[p1_v], s1)
        c0.wait()
        c1.wait()

    return k(pos0, pos1, xf)


def _sc_combine_call(pos0, pos1, ys):
    mesh = plsc.VectorSubcoreMesh(core_axis_name="c", subcore_axis_name="s")

    @functools.partial(
        pl.kernel,
        mesh=mesh,
        out_type=jax.ShapeDtypeStruct((T, D), jnp.float32),
        scratch_types=[
            pltpu.VMEM((CT,), jnp.int32),
            pltpu.VMEM((CT,), jnp.int32),
            pltpu.VMEM((CT, D), jnp.float32),
            pltpu.VMEM((CT, D), jnp.float32),
            pltpu.SemaphoreType.DMA,
            pltpu.SemaphoreType.DMA,
        ],
    )
    def k(p0_hbm, p1_hbm, ys_hbm, out_hbm, p0_v, p1_v, a_v, b_v, s0, s1):
        wid = lax.axis_index("s") * NC + lax.axis_index("c")
        base = wid * TW
        for c in range(TW // CT):
            tb = base + c * CT
            pltpu.sync_copy(p0_hbm.at[pl.ds(tb, CT)], p0_v)
            pltpu.sync_copy(p1_hbm.at[pl.ds(tb, CT)], p1_v)
            cp0 = pltpu.async_copy(ys_hbm.at[p0_v], a_v, s0)
            cp1 = pltpu.async_copy(ys_hbm.at[p1_v], b_v, s1)
            cp0.wait()
            cp1.wait()

            def row(r, carry):
                def col(j, carry2):
                    sl = pl.ds(j * 16, 16)
                    a_v[r, sl] = a_v[r, sl] + b_v[r, sl]
                    return carry2
                return lax.fori_loop(0, D // 16, col, carry)

            lax.fori_loop(0, CT, row, 0)
            pltpu.sync_copy(a_v, out_hbm.at[pl.ds(tb, CT)])

    return k(pos0, pos1, ys)


def kernel(x, Wg, Wfc, Wproj):
    Bv, Tv, Dv = x.shape
    xf = x.reshape(Tv * Bv, Dv)
    gate_probs, rw, ei = _router(xf, Wg)
    wslot, eb, act, chg, pos0, pos1, counts = _dispatch(rw, ei)
    xs = _sc_dispatch_call(pos0, pos1, xf)
    ys = _mlp_call(eb, act, chg, xs, Wfc, Wproj, wslot.reshape(NB, BM, 1))
    out = _sc_combine_call(pos0, pos1, ys)
    freq = counts.astype(jnp.float32) / T
    balance_loss = (gate_probs.mean(axis=0) * freq).sum() * E
    return out.reshape(Bv, Tv, Dv), balance_loss


# speedup vs baseline: 2.2296x; 1.0402x over previous
import functools

import jax
import jax.numpy as jnp
from jax import lax
from jax.experimental import pallas as pl
from jax.experimental.pallas import tpu as pltpu
from jax.experimental.pallas import tpu_sc as plsc

T, D, E, K, H = 2048, 1024, 8, 2, 4096
BM = 256
HTILE = 512
HT = H // HTILE
P = K * T + E * BM
NB = P // BM
NC, NS = 2, 16
NW = NC * NS
ROWS_G = P // NW
CH = 64
TW = T // NW
CT = 32


def _router(xf, Wg):
    gate_logits = xf @ Wg.T
    gate_probs = jax.nn.softmax(gate_logits, axis=-1)
    rw, ei = lax.top_k(gate_probs, K)
    rw = rw / rw.sum(axis=-1, keepdims=True)
    return gate_probs, rw, ei


def _dispatch(rw, ei):
    ee = ei.reshape(-1)
    onehot = (ee[:, None] == jnp.arange(E, dtype=ee.dtype)[None, :])
    incl = jnp.cumsum(onehot.astype(jnp.int32), axis=0)
    rank = jnp.sum(incl * onehot.astype(jnp.int32), axis=1) - 1
    counts = incl[-1]
    padded = ((counts + BM - 1) // BM) * BM
    ends = jnp.cumsum(padded)
    starts = ends - padded
    posf = (starts[ee] + rank).astype(jnp.int32)
    total = ends[-1]
    bstart = jnp.arange(NB, dtype=jnp.int32) * BM
    act = (bstart < total).astype(jnp.int32)
    ebf = jnp.sum(bstart[:, None] >= ends[None, :], axis=1).astype(jnp.int32)
    n_act = total // BM
    eb_last = ebf[jnp.maximum(n_act - 1, 0)]
    eb = jnp.where(act == 1, ebf, eb_last).astype(jnp.int32)
    prev_eb = jnp.concatenate([eb[:1] - 1, eb[:-1]])
    chg = (act * (eb != prev_eb)).astype(jnp.int32)
    pos0 = posf[0::2]
    pos1 = posf[1::2]
    return eb, act, chg, pos0, pos1, counts


def _mlp_body(eb_ref, act_ref, chg_ref, x_ref, wfc_ref, wpr_hbm, o_ref,
              h_ref, wp_v, sem):
    b = pl.program_id(0)

    @pl.when(chg_ref[b] == 1)
    def _():
        pltpu.make_async_copy(wpr_hbm.at[eb_ref[b]], wp_v, sem).start()

    @pl.when(act_ref[b] == 1)
    def _():
        x = x_ref[...]
        for t in range(HT):
            h = lax.dot_general(x, wfc_ref[0, pl.ds(t * HTILE, HTILE), :],
                                (((1,), (1,)), ((), ())),
                                preferred_element_type=jnp.float32)
            h_ref[:, pl.ds(t * HTILE, HTILE)] = jnp.square(jnp.maximum(h, 0.0))

        @pl.when(chg_ref[b] == 1)
        def _():
            pltpu.make_async_copy(wpr_hbm.at[eb_ref[b]], wp_v, sem).wait()

        o_ref[...] = lax.dot_general(h_ref[...], wp_v[...],
                                     (((1,), (1,)), ((), ())),
                                     preferred_element_type=jnp.float32)


def _mlp_call(eb, act, chg, xs, Wfc, Wproj, interpret=False):
    return pl.pallas_call(
        _mlp_body,
        grid_spec=pltpu.PrefetchScalarGridSpec(
            num_scalar_prefetch=3,
            grid=(NB,),
            in_specs=[
                pl.BlockSpec((BM, D), lambda b, eb_r, act_r, chg_r: (b, 0)),
                pl.BlockSpec((1, H, D),
                             lambda b, eb_r, act_r, chg_r: (eb_r[b], 0, 0)),
                pl.BlockSpec(memory_space=pl.ANY),
            ],
            out_specs=pl.BlockSpec((BM, D),
                                   lambda b, eb_r, act_r, chg_r: (b, 0)),
            scratch_shapes=[
                pltpu.VMEM((BM, H), jnp.float32),
                pltpu.VMEM((D, H), jnp.float32),
                pltpu.SemaphoreType.DMA,
            ],
        ),
        out_shape=jax.ShapeDtypeStruct((P, D), jnp.float32),
        compiler_params=pltpu.CompilerParams(
            vmem_limit_bytes=62 * 1024 * 1024),
        interpret=interpret,
    )(eb, act, chg, xs, Wfc, Wproj)


def _sc_dispatch_call(pos0, pos1, xf):
    mesh = plsc.VectorSubcoreMesh(core_axis_name="c", subcore_axis_name="s")

    @functools.partial(
        pl.kernel,
        mesh=mesh,
        out_type=jax.ShapeDtypeStruct((P, D), jnp.float32),
        scratch_types=[
            pltpu.VMEM((TW,), jnp.int32),
            pltpu.VMEM((TW,), jnp.int32),
            pltpu.VMEM((TW, D), jnp.float32),
            pltpu.SemaphoreType.DMA,
            pltpu.SemaphoreType.DMA,
        ],
    )
    def k(p0_hbm, p1_hbm, x_hbm, out_hbm, p0_v, p1_v, rows_v, s0, s1):
        wid = lax.axis_index("s") * NC + lax.axis_index("c")
        tb = wid * TW
        pltpu.sync_copy(p0_hbm.at[pl.ds(tb, TW)], p0_v)
        pltpu.sync_copy(p1_hbm.at[pl.ds(tb, TW)], p1_v)
        pltpu.sync_copy(x_hbm.at[pl.ds(tb, TW)], rows_v)
        c0 = pltpu.async_copy(rows_v, out_hbm.at[p0_v], s0)
        c1 = pltpu.async_copy(rows_v, out_hbm.at[p1_v], s1)
        c0.wait()
        c1.wait()

    return k(pos0, pos1, xf)


def _sc_combine_call(pos0, pos1, w0, w1, ys):
    mesh = plsc.VectorSubcoreMesh(core_axis_name="c", subcore_axis_name="s")

    @functools.partial(
        pl.kernel,
        mesh=mesh,
        out_type=jax.ShapeDtypeStruct((T, D), jnp.float32),
        scratch_types=[
            pltpu.VMEM((CT,), jnp.int32),
            pltpu.VMEM((CT,), jnp.int32),
            pltpu.VMEM((CT, 16), jnp.float32),
            pltpu.VMEM((CT, 16), jnp.float32),
            pltpu.VMEM((CT, D), jnp.float32),
            pltpu.VMEM((CT, D), jnp.float32),
            pltpu.SemaphoreType.DMA,
            pltpu.SemaphoreType.DMA,
        ],
    )
    def k(p0_hbm, p1_hbm, w0_hbm, w1_hbm, ys_hbm, out_hbm,
          p0_v, p1_v, w0_v, w1_v, a_v, b_v, s0, s1):
        wid = lax.axis_index("s") * NC + lax.axis_index("c")
        base = wid * TW
        for c in range(TW // CT):
            tb = base + c * CT
            pltpu.sync_copy(p0_hbm.at[pl.ds(tb, CT)], p0_v)
            pltpu.sync_copy(p1_hbm.at[pl.ds(tb, CT)], p1_v)
            pltpu.sync_copy(w0_hbm.at[pl.ds(tb, CT)], w0_v)
            pltpu.sync_copy(w1_hbm.at[pl.ds(tb, CT)], w1_v)
            cp0 = pltpu.async_copy(ys_hbm.at[p0_v], a_v, s0)
            cp1 = pltpu.async_copy(ys_hbm.at[p1_v], b_v, s1)
            cp0.wait()
            cp1.wait()

            def row(r, carry):
                wa = w0_v[r]
                wb = w1_v[r]

                def col(j, carry2):
                    sl = pl.ds(j * 16, 16)
                    a_v[r, sl] = wa * a_v[r, sl] + wb * b_v[r, sl]
                    return carry2
                return lax.fori_loop(0, D // 16, col, carry)

            lax.fori_loop(0, CT, row, 0)
            pltpu.sync_copy(a_v, out_hbm.at[pl.ds(tb, CT)])

    return k(pos0, pos1, w0, w1, ys)


def kernel(x, Wg, Wfc, Wproj):
    Bv, Tv, Dv = x.shape
    xf = x.reshape(Tv * Bv, Dv)
    gate_probs, rw, ei = _router(xf, Wg)
    eb, act, chg, pos0, pos1, counts = _dispatch(rw, ei)
    xs = _sc_dispatch_call(pos0, pos1, xf)
    ys = _mlp_call(eb, act, chg, xs, Wfc, Wproj)
    w0b = jnp.broadcast_to(rw[:, :1], (T, 16))
    w1b = jnp.broadcast_to(rw[:, 1:], (T, 16))
    out = _sc_combine_call(pos0, pos1, w0b, w1b, ys)
    freq = counts.astype(jnp.float32) / T
    balance_loss = (gate_probs.mean(axis=0) * freq).sum() * E
    return out.reshape(Bv, Tv, Dv), balance_loss
